# baseline (speedup 1.0000x reference)
import functools

import jax
import jax.numpy as jnp
from jax import lax
from jax.experimental import pallas as pl
from jax.experimental.pallas import tpu as pltpu
from jax.experimental.pallas import tpu_sc as plsc

_B = 16384
_NEG = 100
_LANES = 16
_NC = 2
_NS = 16
_NW = _NC * _NS
_ROWS_W = _B // _NW
_WORDS_W = _ROWS_W * _NEG
_GROUPS = _ROWS_W // _LANES
_CHUNKS = -(-_NEG // _LANES)


def _sc_body(sim_hbm, len_hbm, lab_hbm, out_hbm, sim_v, len_v, lab_v, res_v):
    wid = lax.axis_index("s") * _NC + lax.axis_index("c")
    base_row = wid * _ROWS_W
    pltpu.sync_copy(sim_hbm.at[pl.ds(base_row * _NEG, _WORDS_W)],
                    sim_v.at[pl.ds(0, _WORDS_W)])
    pltpu.sync_copy(len_hbm.at[pl.ds(base_row, _ROWS_W)], len_v)
    pltpu.sync_copy(lab_hbm.at[pl.ds(base_row, _ROWS_W)], lab_v)

    lane = lax.iota(jnp.int32, _LANES)
    big = jnp.float32(3e38)
    zero = jnp.zeros((_LANES,), jnp.float32)
    bigv = jnp.full((_LANES,), big)

    def group_body(g, grand):
        sum_vec = zero
        min_vec = bigv
        l_vec = len_v[pl.ds(g * _LANES, _LANES)]
        for r16 in range(_LANES):
            r = g * _LANES + r16
            l_r = l_vec[r16]
            row_sum = zero
            row_min = bigv
            for c in range(_CHUNKS):
                x = sim_v[pl.ds(r * _NEG + c * _LANES, _LANES)]
                m = (lane + (c * _LANES)) < l_r
                xm = jnp.where(m, x, big)
                row_min = jnp.minimum(row_min, xm)
                row_sum = row_sum + jnp.maximum(jnp.float32(0.001) - xm,
                                                jnp.float32(0.0))
            sel = lane == r16
            sum_vec = jnp.where(sel, jnp.sum(row_sum), sum_vec)
            min_vec = jnp.where(sel, jnp.min(row_min), min_vec)
        l_f = l_vec.astype(jnp.float32)
        lab = lab_v[pl.ds(g * _LANES, _LANES)]
        mean = sum_vec / l_f
        u = min_vec + jnp.float32(0.15)
        mis = jnp.where(u >= 0, u, u * jnp.float32(0.01))
        return grand + jnp.where(lab == 1, mean, mis)

    grand = lax.fori_loop(0, _GROUPS, group_body, zero)
    res_v[...] = jnp.where(lane == 0, jnp.sum(grand), jnp.float32(0.0))
    pltpu.sync_copy(res_v, out_hbm.at[wid])


@jax.jit
def _mnloss_sc(sim_flat, lengths, labels):
    mesh = plsc.VectorSubcoreMesh(core_axis_name="c", subcore_axis_name="s")
    run = functools.partial(
        pl.kernel,
        mesh=mesh,
        compiler_params=pltpu.CompilerParams(needs_layout_passes=False),
        out_type=jax.ShapeDtypeStruct((_NW, _LANES), jnp.float32),
        scratch_types=[
            pltpu.VMEM((_WORDS_W + _LANES,), jnp.float32),
            pltpu.VMEM((_ROWS_W,), jnp.int32),
            pltpu.VMEM((_ROWS_W,), jnp.int32),
            pltpu.VMEM((_LANES,), jnp.float32),
        ],
    )(_sc_body)
    return run(sim_flat, lengths, labels)


def kernel(sim_neg, train_mn_label, mn_length):
    partials = _mnloss_sc(sim_neg.reshape(-1), mn_length, train_mn_label)
    return jnp.sum(partials).reshape(1)
